# baseline (speedup 1.0000x reference)
import jax
import jax.numpy as jnp
from jax import lax
from jax.experimental import pallas as pl
from jax.experimental.pallas import tpu as pltpu
from jax.experimental.pallas import tpu_sc as plsc

B, S, D = 4, 8192, 1024
NW = 32
R = 16
HR = R // 2
S_PER_W = S // NW
SCHUNKS = S_PER_W // R
NBUF = 4


def _sc_body(x_hbm, pe_hbm, out_hbm, pe_bufs, xbufs, pe_sems, lsems, ssems):
    nc = 2
    wid = lax.axis_index("s") * nc + lax.axis_index("c")
    s_base = wid * S_PER_W

    def s0(c):
        return s_base + c * R

    def start_load(c, b, p):
        pltpu.async_copy(x_hbm.at[b, pl.ds(s0(c), R)], xbufs[p], lsems[p])

    def wait_load(c, b, p):
        pltpu.make_async_copy(
            x_hbm.at[b, pl.ds(s0(c), R)], xbufs[p], lsems[p]).wait()

    def start_store_half(c, b, p, h):
        pltpu.async_copy(xbufs[p].at[pl.ds(h * HR, HR)],
                         out_hbm.at[b, pl.ds(s0(c) + h * HR, HR)], ssems[p])

    def wait_store(c, b, p):
        pltpu.make_async_copy(
            xbufs[p], out_hbm.at[b, pl.ds(s0(c), R)], ssems[p]).wait()

    def start_pe_load(c, e):
        pltpu.async_copy(pe_hbm.at[pl.ds(s0(c), R)], pe_bufs[e], pe_sems[e])

    def wait_pe_load(c, e):
        pltpu.make_async_copy(
            pe_hbm.at[pl.ds(s0(c), R)], pe_bufs[e], pe_sems[e]).wait()

    start_pe_load(0, 0)
    start_load(0, 0, 0)
    start_load(0, 1, 1)

    def chunk_body(c, e):
        wait_pe_load(c, e)

        @pl.when(c < SCHUNKS - 1)
        def _():
            start_pe_load(c + 1, 1 - e)

        for b in range(B):
            wait_load(c, b, b)
            for h in range(2):

                @plsc.parallel_loop(0, HR, unroll=1)
                def vadd_row(r, _b=b, _e=e, _h=h):
                    @plsc.parallel_loop(0, D // 16, unroll=8)
                    def vadd(j):
                        plsc.addupdate(
                            xbufs[_b].at[_h * HR + r, pl.ds(j * 16, 16)],
                            pe_bufs[_e][_h * HR + r, pl.ds(j * 16, 16)])

                start_store_half(c, b, b, h)

            q = (b + 2) % NBUF
            if b < 2:
                @pl.when(c > 0)
                def _():
                    wait_store(c - 1, b + 2, q)
                start_load(c, b + 2, q)
            else:
                @pl.when(c < SCHUNKS - 1)
                def _():
                    wait_store(c, b - 2, q)
                    start_load(c + 1, b - 2, q)

    def schunk_pair(cc, carry):
        chunk_body(cc * 2, 0)
        chunk_body(cc * 2 + 1, 1)
        return carry

    lax.fori_loop(0, SCHUNKS // 2, schunk_pair, 0)

    wait_store(SCHUNKS - 1, 2, 2)
    wait_store(SCHUNKS - 1, 3, 3)


def kernel(x, pe_weight):
    return pl.kernel(
        _sc_body,
        out_type=jax.ShapeDtypeStruct((B, S, D), jnp.float32),
        mesh=plsc.VectorSubcoreMesh(core_axis_name="c", subcore_axis_name="s"),
        scratch_types=[
            [pltpu.VMEM((R, D), jnp.float32) for _ in range(2)],
            [pltpu.VMEM((R, D), jnp.float32) for _ in range(NBUF)],
            [pltpu.SemaphoreType.DMA for _ in range(2)],
            [pltpu.SemaphoreType.DMA for _ in range(NBUF)],
            [pltpu.SemaphoreType.DMA for _ in range(NBUF)],
        ],
        compiler_params=pltpu.CompilerParams(use_tc_tiling_on_sc=True),
    )(x, pe_weight)


# speedup vs baseline: 1.0609x; 1.0609x over previous
import jax
import jax.numpy as jnp
from jax import lax
from jax.experimental import pallas as pl
from jax.experimental.pallas import tpu as pltpu
from jax.experimental.pallas import tpu_sc as plsc

B, S, D = 4, 8192, 1024
NW = 32
R = 8
S_PER_W = S // NW
SCHUNKS = S_PER_W // R
NBUF = 8


def _sc_body(x_hbm, pe_hbm, out_hbm, pe_bufs, xbufs, pe_sems, lsems, ssems):
    nc = 2
    wid = lax.axis_index("s") * nc + lax.axis_index("c")
    s_base = wid * S_PER_W

    def s0(c):
        return s_base + c * R

    def start_load(c, b, p):
        pltpu.async_copy(x_hbm.at[b, pl.ds(s0(c), R)], xbufs[p], lsems[p])

    def wait_load(c, b, p):
        pltpu.make_async_copy(
            x_hbm.at[b, pl.ds(s0(c), R)], xbufs[p], lsems[p]).wait()

    def start_store(c, b, p):
        pltpu.async_copy(xbufs[p], out_hbm.at[b, pl.ds(s0(c), R)], ssems[p])

    def wait_store(c, b, p):
        pltpu.make_async_copy(
            xbufs[p], out_hbm.at[b, pl.ds(s0(c), R)], ssems[p]).wait()

    def start_pe_load(c, e):
        pltpu.async_copy(pe_hbm.at[pl.ds(s0(c), R)], pe_bufs[e], pe_sems[e])

    def wait_pe_load(c, e):
        pltpu.make_async_copy(
            pe_hbm.at[pl.ds(s0(c), R)], pe_bufs[e], pe_sems[e]).wait()

    start_pe_load(0, 0)
    for b in range(B):
        start_load(0, b, b)

    def chunk_body(c, e):
        wait_pe_load(c, e)

        @pl.when(c < SCHUNKS - 1)
        def _():
            start_pe_load(c + 1, 1 - e)

        for b in range(B):
            mine = 4 * e + b
            other = 4 * (1 - e) + b
            wait_load(c, b, mine)

            @plsc.parallel_loop(0, R, unroll=1)
            def vadd_row(r, _b=b, _e=e):
                @plsc.parallel_loop(0, D // 16, unroll=8)
                def vadd(j):
                    plsc.addupdate(xbufs[4 * _e + _b].at[r, pl.ds(j * 16, 16)],
                                   pe_bufs[_e][r, pl.ds(j * 16, 16)])

            start_store(c, b, mine)

            @pl.when(c < SCHUNKS - 1)
            def _():
                @pl.when(c > 0)
                def _():
                    wait_store(c - 1, b, other)
                start_load(c + 1, b, other)

    def schunk_pair(cc, carry):
        chunk_body(cc * 2, 0)
        chunk_body(cc * 2 + 1, 1)
        return carry

    lax.fori_loop(0, SCHUNKS // 2, schunk_pair, 0)

    for b in range(B):
        wait_store(SCHUNKS - 2, b, b)
        wait_store(SCHUNKS - 1, b, 4 + b)


def kernel(x, pe_weight):
    return pl.kernel(
        _sc_body,
        out_type=jax.ShapeDtypeStruct((B, S, D), jnp.float32),
        mesh=plsc.VectorSubcoreMesh(core_axis_name="c", subcore_axis_name="s"),
        scratch_types=[
            [pltpu.VMEM((R, D), jnp.float32) for _ in range(2)],
            [pltpu.VMEM((R, D), jnp.float32) for _ in range(NBUF)],
            [pltpu.SemaphoreType.DMA for _ in range(2)],
            [pltpu.SemaphoreType.DMA for _ in range(NBUF)],
            [pltpu.SemaphoreType.DMA for _ in range(NBUF)],
        ],
        compiler_params=pltpu.CompilerParams(use_tc_tiling_on_sc=True),
    )(x, pe_weight)
